# baseline (speedup 1.0000x reference)
import functools

import jax
import jax.numpy as jnp
from jax import lax
from jax.experimental import pallas as pl
from jax.experimental.pallas import tpu as pltpu
from jax.experimental.pallas import tpu_sc as plsc

N = 10000
NP = 10240
E = 160000
D = 256
HD = D // 2
NC = 2
NS = 16
EPT = E // NS
K = 128
NCHUNK = EPT // K
KT = EPT - NCHUNK * K
ROWS = NP // NS

_mesh = plsc.VectorSubcoreMesh(
    core_axis_name="c", subcore_axis_name="s", num_cores=NC, num_subcores=NS
)


@functools.partial(
    pl.kernel,
    out_type=jax.ShapeDtypeStruct((NC, NP, HD), jnp.float32),
    mesh=_mesh,
    scratch_types=[
        pltpu.VMEM((NCHUNK, K), jnp.int32),
        pltpu.VMEM((NCHUNK, K), jnp.int32),
        pltpu.VMEM((1, KT), jnp.int32),
        pltpu.VMEM((1, KT), jnp.int32),
        pltpu.VMEM((K, HD), jnp.float32),
        pltpu.VMEM_SHARED((NP, HD), jnp.float32),
        pltpu.SemaphoreType.DMA,
    ],
)
def _sc_segsum(
    feat_hbm, fr2_hbm, src_hbm, dst_hbm, srct_hbm, dstt_hbm, out_hbm,
    src_v, dst_v, srct_v, dstt_v, buf, acc, sem,
):
    c = lax.axis_index("c")
    s = lax.axis_index("s")
    base = s * ROWS
    @pl.when(s < NS - 1)
    def _():
        pltpu.sync_copy(
            feat_hbm.at[pl.ds(base, ROWS), pl.ds(c * HD, HD)],
            acc.at[pl.ds(base, ROWS)],
        )

    @pl.when(s == NS - 1)
    def _():
        last = (NS - 1) * ROWS
        pltpu.sync_copy(
            feat_hbm.at[pl.ds(last, N - last), pl.ds(c * HD, HD)],
            acc.at[pl.ds(last, N - last)],
        )

    pltpu.sync_copy(src_hbm.at[c, s], src_v)
    pltpu.sync_copy(dst_hbm.at[s], dst_v)
    pltpu.sync_copy(srct_hbm.at[c, s], srct_v)
    pltpu.sync_copy(dstt_hbm.at[s], dstt_v)
    plsc.subcore_barrier()

    def chunk(j, carry):
        pltpu.async_copy(fr2_hbm.at[src_v.at[j]], buf, sem).wait()
        pltpu.sync_copy(buf, acc.at[dst_v.at[j]], add=True)
        return carry

    lax.fori_loop(0, NCHUNK, chunk, 0)
    pltpu.async_copy(fr2_hbm.at[srct_v.at[0]], buf.at[pl.ds(0, KT)], sem).wait()
    pltpu.sync_copy(buf.at[pl.ds(0, KT)], acc.at[dstt_v.at[0]], add=True)
    plsc.subcore_barrier()
    pltpu.sync_copy(
        acc.at[pl.ds(base, ROWS)], out_hbm.at[c, pl.ds(base, ROWS)]
    )


BN = 1000


def _tc_mlp(h2_ref, wc_ref, bc_ref, wm_ref, bm_ref, out_ref):
    h = jnp.concatenate([h2_ref[0], h2_ref[1]], axis=-1)
    coff = jax.nn.sigmoid(
        jnp.dot(h, wc_ref[...], preferred_element_type=jnp.float32) + bc_ref[...]
    )
    msg = jnp.tanh(
        jnp.dot(h, wm_ref[...], preferred_element_type=jnp.float32) + bm_ref[...]
    )
    out_ref[...] = coff * msg


def kernel(feat, edge_index, W_coff, b_coff, W_msg, b_msg):
    src = edge_index[0]
    dst = edge_index[1]
    fr2 = feat.reshape(2 * N, HD)
    src_p = 2 * src.reshape(NS, EPT)
    dst_p = dst.reshape(NS, EPT)
    src_m = src_p[:, : NCHUNK * K].reshape(NS, NCHUNK, K)
    src_t = src_p[:, NCHUNK * K :].reshape(NS, 1, KT)
    src2 = jnp.stack([src_m, src_m + 1])
    src2t = jnp.stack([src_t, src_t + 1])
    dst_m = dst_p[:, : NCHUNK * K].reshape(NS, NCHUNK, K)
    dst_t = dst_p[:, NCHUNK * K :].reshape(NS, 1, KT)

    h2 = _sc_segsum(feat, fr2, src2, dst_m, src2t, dst_t)

    out = pl.pallas_call(
        _tc_mlp,
        grid=(N // BN,),
        in_specs=[
            pl.BlockSpec((NC, BN, HD), lambda i: (0, i, 0)),
            pl.BlockSpec((D, D), lambda i: (0, 0)),
            pl.BlockSpec((1, D), lambda i: (0, 0)),
            pl.BlockSpec((D, D), lambda i: (0, 0)),
            pl.BlockSpec((1, D), lambda i: (0, 0)),
        ],
        out_specs=pl.BlockSpec((BN, D), lambda i: (i, 0)),
        out_shape=jax.ShapeDtypeStruct((N, D), jnp.float32),
    )(h2, W_coff, b_coff.reshape(1, D), W_msg, b_msg.reshape(1, D))
    return out


# speedup vs baseline: 1.0121x; 1.0121x over previous
import functools

import jax
import jax.numpy as jnp
from jax import lax
from jax.experimental import pallas as pl
from jax.experimental.pallas import tpu as pltpu
from jax.experimental.pallas import tpu_sc as plsc

N = 10000
NP = 10240
E = 160000
D = 256
HD = D // 2
NC = 2
NS = 16
EPT = E // NS
K = 128
NCHUNK = EPT // K
KT = EPT - NCHUNK * K
ROWS = NP // NS

_mesh = plsc.VectorSubcoreMesh(
    core_axis_name="c", subcore_axis_name="s", num_cores=NC, num_subcores=NS
)


@functools.partial(
    pl.kernel,
    out_type=jax.ShapeDtypeStruct((NC, NP, HD), jnp.float32),
    mesh=_mesh,
    scratch_types=[
        pltpu.VMEM((NCHUNK, K), jnp.int32),
        pltpu.VMEM((NCHUNK, K), jnp.int32),
        pltpu.VMEM((1, KT), jnp.int32),
        pltpu.VMEM((1, KT), jnp.int32),
        pltpu.VMEM((K, HD), jnp.float32),
        pltpu.VMEM_SHARED((NP, HD), jnp.float32),
        pltpu.SemaphoreType.DMA,
    ],
)
def _sc_segsum(
    feat_hbm, fr2_hbm, src_hbm, dst_hbm, srct_hbm, dstt_hbm, out_hbm,
    src_v, dst_v, srct_v, dstt_v, buf, acc, sem,
):
    c = lax.axis_index("c")
    s = lax.axis_index("s")
    base = s * ROWS
    @pl.when(s < NS - 1)
    def _():
        pltpu.sync_copy(
            feat_hbm.at[pl.ds(base, ROWS), pl.ds(c * HD, HD)],
            acc.at[pl.ds(base, ROWS)],
        )

    @pl.when(s == NS - 1)
    def _():
        last = (NS - 1) * ROWS
        pltpu.sync_copy(
            feat_hbm.at[pl.ds(last, N - last), pl.ds(c * HD, HD)],
            acc.at[pl.ds(last, N - last)],
        )

    pltpu.sync_copy(src_hbm.at[c, s], src_v)
    pltpu.sync_copy(dst_hbm.at[s], dst_v)
    pltpu.sync_copy(srct_hbm.at[c, s], srct_v)
    pltpu.sync_copy(dstt_hbm.at[s], dstt_v)
    plsc.subcore_barrier()

    def chunk(j, carry):
        pltpu.async_copy(fr2_hbm.at[src_v.at[j]], buf, sem).wait()
        pltpu.sync_copy(buf, acc.at[dst_v.at[j]], add=True)
        return carry

    lax.fori_loop(0, NCHUNK, chunk, 0)
    pltpu.async_copy(fr2_hbm.at[srct_v.at[0]], buf.at[pl.ds(0, KT)], sem).wait()
    pltpu.sync_copy(buf.at[pl.ds(0, KT)], acc.at[dstt_v.at[0]], add=True)
    plsc.subcore_barrier()
    pltpu.sync_copy(
        acc.at[pl.ds(base, ROWS)], out_hbm.at[c, pl.ds(base, ROWS)]
    )


BN = 2000


def _tc_mlp(h2_ref, wc_ref, bc_ref, wm_ref, bm_ref, out_ref):
    h = jnp.concatenate([h2_ref[0], h2_ref[1]], axis=-1)
    coff = jax.nn.sigmoid(
        jnp.dot(h, wc_ref[...], preferred_element_type=jnp.float32) + bc_ref[...]
    )
    msg = jnp.tanh(
        jnp.dot(h, wm_ref[...], preferred_element_type=jnp.float32) + bm_ref[...]
    )
    out_ref[...] = coff * msg


def kernel(feat, edge_index, W_coff, b_coff, W_msg, b_msg):
    src = edge_index[0]
    dst = edge_index[1]
    fr2 = feat.reshape(2 * N, HD)
    src_p = 2 * src.reshape(NS, EPT)
    dst_p = dst.reshape(NS, EPT)
    src_m = src_p[:, : NCHUNK * K].reshape(NS, NCHUNK, K)
    src_t = src_p[:, NCHUNK * K :].reshape(NS, 1, KT)
    src2 = jnp.stack([src_m, src_m + 1])
    src2t = jnp.stack([src_t, src_t + 1])
    dst_m = dst_p[:, : NCHUNK * K].reshape(NS, NCHUNK, K)
    dst_t = dst_p[:, NCHUNK * K :].reshape(NS, 1, KT)

    h2 = _sc_segsum(feat, fr2, src2, dst_m, src2t, dst_t)

    out = pl.pallas_call(
        _tc_mlp,
        grid=(N // BN,),
        in_specs=[
            pl.BlockSpec((NC, BN, HD), lambda i: (0, i, 0)),
            pl.BlockSpec((D, D), lambda i: (0, 0)),
            pl.BlockSpec((1, D), lambda i: (0, 0)),
            pl.BlockSpec((D, D), lambda i: (0, 0)),
            pl.BlockSpec((1, D), lambda i: (0, 0)),
        ],
        out_specs=pl.BlockSpec((BN, D), lambda i: (i, 0)),
        out_shape=jax.ShapeDtypeStruct((N, D), jnp.float32),
    )(h2, W_coff, b_coff.reshape(1, D), W_msg, b_msg.reshape(1, D))
    return out


# speedup vs baseline: 1.0170x; 1.0048x over previous
import functools

import jax
import jax.numpy as jnp
from jax import lax
from jax.experimental import pallas as pl
from jax.experimental.pallas import tpu as pltpu
from jax.experimental.pallas import tpu_sc as plsc

N = 10000
NP = 10240
E = 160000
D = 256
HD = D // 2
NC = 2
NS = 16
EPT = E // NS
K = 128
NCHUNK = EPT // K
KT = EPT - NCHUNK * K
ROWS = NP // NS

_mesh = plsc.VectorSubcoreMesh(
    core_axis_name="c", subcore_axis_name="s", num_cores=NC, num_subcores=NS
)


@functools.partial(
    pl.kernel,
    out_type=jax.ShapeDtypeStruct((NC, NP, HD), jnp.float32),
    mesh=_mesh,
    scratch_types=[
        pltpu.VMEM((NCHUNK, K), jnp.int32),
        pltpu.VMEM((NCHUNK, K), jnp.int32),
        pltpu.VMEM((1, KT), jnp.int32),
        pltpu.VMEM((1, KT), jnp.int32),
        pltpu.VMEM((K, HD), jnp.float32),
        pltpu.VMEM_SHARED((NP, HD), jnp.float32),
        pltpu.SemaphoreType.DMA,
    ],
)
def _sc_segsum(
    feat_hbm, fr2_hbm, src_hbm, dst_hbm, srct_hbm, dstt_hbm, out_hbm,
    src_v, dst_v, srct_v, dstt_v, buf, acc, sem,
):
    c = lax.axis_index("c")
    s = lax.axis_index("s")
    base = s * ROWS
    @pl.when(s < NS - 1)
    def _():
        pltpu.sync_copy(
            feat_hbm.at[pl.ds(base, ROWS), pl.ds(c * HD, HD)],
            acc.at[pl.ds(base, ROWS)],
        )

    @pl.when(s == NS - 1)
    def _():
        last = (NS - 1) * ROWS
        pltpu.sync_copy(
            feat_hbm.at[pl.ds(last, N - last), pl.ds(c * HD, HD)],
            acc.at[pl.ds(last, N - last)],
        )

    pltpu.sync_copy(src_hbm.at[c, s], src_v)
    pltpu.sync_copy(dst_hbm.at[s], dst_v)
    pltpu.sync_copy(srct_hbm.at[c, s], srct_v)
    pltpu.sync_copy(dstt_hbm.at[s], dstt_v)
    plsc.subcore_barrier()

    def chunk(j, carry):
        pltpu.async_copy(fr2_hbm.at[src_v.at[j]], buf, sem).wait()
        pltpu.sync_copy(buf, acc.at[dst_v.at[j]], add=True)
        return carry

    lax.fori_loop(0, NCHUNK, chunk, 0)
    pltpu.async_copy(fr2_hbm.at[srct_v.at[0]], buf.at[pl.ds(0, KT)], sem).wait()
    pltpu.sync_copy(buf.at[pl.ds(0, KT)], acc.at[dstt_v.at[0]], add=True)
    plsc.subcore_barrier()
    pltpu.sync_copy(
        acc.at[pl.ds(base, ROWS)], out_hbm.at[c, pl.ds(base, ROWS)]
    )


BN = 5000


def _tc_mlp(h2_ref, wc_ref, bc_ref, wm_ref, bm_ref, out_ref):
    h = jnp.concatenate([h2_ref[0], h2_ref[1]], axis=-1)
    coff = jax.nn.sigmoid(
        jnp.dot(h, wc_ref[...], preferred_element_type=jnp.float32) + bc_ref[...]
    )
    msg = jnp.tanh(
        jnp.dot(h, wm_ref[...], preferred_element_type=jnp.float32) + bm_ref[...]
    )
    out_ref[...] = coff * msg


def kernel(feat, edge_index, W_coff, b_coff, W_msg, b_msg):
    src = edge_index[0]
    dst = edge_index[1]
    fr2 = feat.reshape(2 * N, HD)
    src_p = 2 * src.reshape(NS, EPT)
    dst_p = dst.reshape(NS, EPT)
    src_m = src_p[:, : NCHUNK * K].reshape(NS, NCHUNK, K)
    src_t = src_p[:, NCHUNK * K :].reshape(NS, 1, KT)
    src2 = jnp.stack([src_m, src_m + 1])
    src2t = jnp.stack([src_t, src_t + 1])
    dst_m = dst_p[:, : NCHUNK * K].reshape(NS, NCHUNK, K)
    dst_t = dst_p[:, NCHUNK * K :].reshape(NS, 1, KT)

    h2 = _sc_segsum(feat, fr2, src2, dst_m, src2t, dst_t)

    out = pl.pallas_call(
        _tc_mlp,
        grid=(N // BN,),
        in_specs=[
            pl.BlockSpec((NC, BN, HD), lambda i: (0, i, 0)),
            pl.BlockSpec((D, D), lambda i: (0, 0)),
            pl.BlockSpec((1, D), lambda i: (0, 0)),
            pl.BlockSpec((D, D), lambda i: (0, 0)),
            pl.BlockSpec((1, D), lambda i: (0, 0)),
        ],
        out_specs=pl.BlockSpec((BN, D), lambda i: (i, 0)),
        out_shape=jax.ShapeDtypeStruct((N, D), jnp.float32),
    )(h2, W_coff, b_coff.reshape(1, D), W_msg, b_msg.reshape(1, D))
    return out
